# Initial kernel scaffold; baseline (speedup 1.0000x reference)
import jax
import jax.numpy as jnp
from jax.experimental import pallas as pl


def kernel(x, table_en, table_zh, table_jp):
    raise NotImplementedError("write your pallas kernel here")


# speedup vs baseline: 4.6235x; 4.6235x over previous
import functools

import jax
import jax.numpy as jnp
from jax import lax
from jax.experimental import pallas as pl
from jax.experimental.pallas import tpu as pltpu
from jax.experimental.pallas import tpu_sc as plsc

DIM = 64
CHUNK = 128


@functools.cache
def _make_gather(B, V, nw, nc, nchunks):
    mesh = plsc.VectorSubcoreMesh(core_axis_name="c", subcore_axis_name="s")

    @functools.partial(
        pl.kernel,
        mesh=mesh,
        compiler_params=pltpu.CompilerParams(use_tc_tiling_on_sc=False),
        out_type=jax.ShapeDtypeStruct((B, DIM), jnp.float32),
        scratch_types=[
            pltpu.VMEM((nchunks, CHUNK), jnp.int32),
            pltpu.VMEM((CHUNK, DIM), jnp.float32),
            pltpu.SemaphoreType.DMA,
        ],
    )
    def gather_kernel(table_hbm, idx_hbm, out_hbm, idx_v, rows_v, sem):
        wid = lax.axis_index("s") * nc + lax.axis_index("c")
        base = wid * nchunks * CHUNK
        pltpu.sync_copy(idx_hbm.at[wid], idx_v)

        def body(j, carry):
            pltpu.async_copy(table_hbm.at[idx_v.at[j]], rows_v, sem).wait()
            pltpu.sync_copy(rows_v, out_hbm.at[pl.ds(base + j * CHUNK, CHUNK)])
            return carry

        lax.fori_loop(0, nchunks, body, 0)

    return gather_kernel


def kernel(x, table_en, table_zh, table_jp):
    table = jnp.concatenate([table_en, table_zh, table_jp], axis=0)
    info = plsc.get_sparse_core_info()
    nw = info.num_cores * info.num_subcores
    B = x.size
    nchunks = B // (nw * CHUNK)
    idx3 = x.reshape(nw, nchunks, CHUNK)
    out = _make_gather(B, table.shape[0], nw, info.num_cores, nchunks)(table, idx3)
    return out.reshape(x.shape + (DIM,))


# speedup vs baseline: 4.7602x; 1.0296x over previous
import functools

import jax
import jax.numpy as jnp
from jax import lax
from jax.experimental import pallas as pl
from jax.experimental.pallas import tpu as pltpu
from jax.experimental.pallas import tpu_sc as plsc

DIM = 64
CHUNK = 128
NBUF = 5


@functools.cache
def _make_gather(B, V, nw, nc, nchunks):
    mesh = plsc.VectorSubcoreMesh(core_axis_name="c", subcore_axis_name="s")
    assert nchunks % NBUF == 0

    @functools.partial(
        pl.kernel,
        mesh=mesh,
        compiler_params=pltpu.CompilerParams(use_tc_tiling_on_sc=False),
        out_type=jax.ShapeDtypeStruct((B, DIM), jnp.float32),
        scratch_types=[
            pltpu.VMEM((nchunks, CHUNK), jnp.int32),
            pltpu.VMEM((NBUF, CHUNK, DIM), jnp.float32),
            pltpu.SemaphoreType.DMA((NBUF,)),
            pltpu.SemaphoreType.DMA((NBUF,)),
        ],
    )
    def gather_kernel(table_hbm, idx_hbm, out_hbm, idx_v, rows_v, gsem, ssem):
        wid = lax.axis_index("s") * nc + lax.axis_index("c")
        base = wid * nchunks * CHUNK
        pltpu.sync_copy(idx_hbm.at[wid], idx_v)

        for b in range(NBUF):
            pltpu.async_copy(table_hbm.at[idx_v.at[b]], rows_v.at[b], gsem.at[b])

        def body(jj, carry):
            j0 = jj * NBUF
            for b in range(NBUF):
                j = j0 + b
                pltpu.make_async_copy(
                    table_hbm.at[idx_v.at[j]], rows_v.at[b], gsem.at[b]
                ).wait()
                pltpu.async_copy(
                    rows_v.at[b], out_hbm.at[pl.ds(base + j * CHUNK, CHUNK)], ssem.at[b]
                )
            for b in range(NBUF):
                jn = j0 + NBUF + b
                pltpu.make_async_copy(
                    rows_v.at[b], out_hbm.at[pl.ds(base, CHUNK)], ssem.at[b]
                ).wait()

                @pl.when(jn < nchunks)
                def _():
                    pltpu.async_copy(table_hbm.at[idx_v.at[jn]], rows_v.at[b], gsem.at[b])

            return carry

        lax.fori_loop(0, nchunks // NBUF, body, 0)

    return gather_kernel


def kernel(x, table_en, table_zh, table_jp):
    table = jnp.concatenate([table_en, table_zh, table_jp], axis=0)
    info = plsc.get_sparse_core_info()
    nw = info.num_cores * info.num_subcores
    B = x.size
    nchunks = B // (nw * CHUNK)
    idx3 = x.reshape(nw, nchunks, CHUNK)
    out = _make_gather(B, table.shape[0], nw, info.num_cores, nchunks)(table, idx3)
    return out.reshape(x.shape + (DIM,))
